# baseline (speedup 1.0000x reference)
import functools
import jax
import jax.numpy as jnp
from jax import lax
from jax.experimental import pallas as pl
from jax.experimental.pallas import tpu as pltpu
from jax.experimental.pallas import tpu_sc as plsc

_CTXW = 20
_D = 128
_H = 128
_V = 100000
_R = 4096
_NB = (_V + _R - 1) // _R
_S = 5
_P1 = (_NB + _S - 1) // _S
_OFFS = [min(k * _P1, _NB) for k in range(_S + 1)]


_GPAD = 32
_GW = 4
_GB = _GPAD // _GW


def _sc_gather_kernel(tab_ref, idx_ref, out_ref, idx_v, rows_v, sem):
    wid = lax.axis_index("s") * 2 + lax.axis_index("c")

    @pl.when(wid < _GW)
    def _do():
        base = wid * _GB
        pltpu.sync_copy(idx_ref.at[pl.ds(base, _GB)], idx_v)
        pltpu.async_copy(tab_ref.at[idx_v], rows_v, sem).wait()
        pltpu.sync_copy(rows_v, out_ref.at[pl.ds(base, _GB), :])


def _l1_kernel(e_ref, w1_ref, b1_ref, h_ref):
    e = e_ref[:, :_CTXW * _D]
    h = jnp.dot(e, w1_ref[...].T, preferred_element_type=jnp.float32)
    h_ref[...] = jnp.maximum(h + b1_ref[...], 0.0)


def _l2_kernel(h_ref, b2_ref, *refs):
    w2_refs = refs[:_S]
    out_ref = refs[_S]
    logits_ref, m_ref, s_ref = refs[_S + 1:]
    t = pl.program_id(0)

    @pl.when(t == 0)
    def _init():
        m_ref[0, 0] = -jnp.inf
        s_ref[0, 0] = 0.0

    @pl.when(t < _P1)
    def _stream():
        h = h_ref[...]
        for k in range(_S):
            cnt = _OFFS[k + 1] - _OFFS[k]

            @pl.when(t < cnt)
            def _do(k=k):
                b = _OFFS[k] + t
                logits = jnp.dot(h, w2_refs[k][...].T,
                                 preferred_element_type=jnp.float32)
                logits = logits + b2_ref[:, pl.ds(b * _R, _R)]
                col = b * _R + jax.lax.broadcasted_iota(jnp.int32, (1, _R), 1)
                logits = jnp.where(col < _V, logits, -jnp.inf)
                logits_ref[:, pl.ds(b * _R, _R)] = logits

                tile_max = jnp.max(logits)
                m_old = m_ref[0, 0]
                m_new = jnp.maximum(m_old, tile_max)
                s_ref[0, 0] = (s_ref[0, 0] * jnp.exp(m_old - m_new)
                               + jnp.sum(jnp.exp(logits - m_new)))
                m_ref[0, 0] = m_new

    @pl.when(t == _P1 - 1)
    def _fin():
        m_ref[0, 0] = m_ref[0, 0] + jnp.log(s_ref[0, 0])

    @pl.when(t == _P1)
    def _emit():
        out_ref[...] = logits_ref[:, :_V] - m_ref[0, 0]


def kernel(inputs, table, W1, b1, W2, b2):
    idx = inputs.astype(jnp.int32)
    b1r = b1.reshape(1, _H)
    b2r = jnp.pad(b2.reshape(1, _V), ((0, 0), (0, _NB * _R - _V)))

    idx32 = jnp.pad(idx, (0, _GPAD - _CTXW))
    embeds = pl.kernel(
        _sc_gather_kernel,
        out_type=jax.ShapeDtypeStruct((_GPAD, _D), jnp.float32),
        mesh=plsc.VectorSubcoreMesh(core_axis_name="c", subcore_axis_name="s"),
        scratch_types=[
            pltpu.VMEM((_GB,), jnp.int32),
            pltpu.VMEM((_GB, _D), jnp.float32),
            pltpu.SemaphoreType.DMA,
        ],
    )(table, idx32)
    e_flat = embeds.reshape(1, _GPAD * _D)

    h = pl.pallas_call(
        _l1_kernel,
        in_specs=[
            pl.BlockSpec(memory_space=pltpu.VMEM),
            pl.BlockSpec(memory_space=pltpu.VMEM),
            pl.BlockSpec(memory_space=pltpu.VMEM),
        ],
        out_specs=pl.BlockSpec(memory_space=pltpu.VMEM),
        out_shape=jax.ShapeDtypeStruct((1, _H), jnp.float32),
    )(e_flat, W1, b1r)

    def _w2_map(k):
        lo, hi = _OFFS[k], _OFFS[k + 1] - 1
        return lambda t: (jnp.clip(lo + t, lo, hi), 0)

    out = pl.pallas_call(
        _l2_kernel,
        grid=(_P1 + 1,),
        in_specs=(
            [pl.BlockSpec((1, _H), lambda t: (0, 0)),
             pl.BlockSpec((1, _NB * _R), lambda t: (0, 0))]
            + [pl.BlockSpec((_R, _D), _w2_map(k)) for k in range(_S)]
        ),
        out_specs=pl.BlockSpec((1, _V), lambda t: (0, 0)),
        out_shape=jax.ShapeDtypeStruct((1, _V), jnp.float32),
        scratch_shapes=[
            pltpu.VMEM((1, _NB * _R), jnp.float32),
            pltpu.SMEM((1, 1), jnp.float32),
            pltpu.SMEM((1, 1), jnp.float32),
        ],
    )(h, b2r, *([W2] * _S))

    return out


# speedup vs baseline: 1.7491x; 1.7491x over previous
import jax
import jax.numpy as jnp
from jax.experimental import pallas as pl
from jax.experimental.pallas import tpu as pltpu

_CTXW = 20
_D = 128
_H = 128
_V = 100000
_R = 4224
_NB = (_V + _R - 1) // _R
_S = 6
_P1 = (_NB + _S - 1) // _S
_OFFS = [min(k * _P1, _NB) for k in range(_S + 1)]


def _l1_kernel(idx_ref, tab_ref, w1_ref, b1_ref, h_ref, e_ref, sems):
    for j in range(_CTXW):
        pltpu.make_async_copy(
            tab_ref.at[pl.ds(idx_ref[j], 1), :],
            e_ref.at[:, pl.ds(j * _D, _D)],
            sems.at[j],
        ).start()
    for j in range(_CTXW):
        pltpu.make_async_copy(
            tab_ref.at[pl.ds(idx_ref[j], 1), :],
            e_ref.at[:, pl.ds(j * _D, _D)],
            sems.at[j],
        ).wait()
    h = jnp.dot(e_ref[...], w1_ref[...].T, preferred_element_type=jnp.float32)
    h_ref[...] = jnp.maximum(h + b1_ref[...], 0.0)


def _l2_kernel(h_ref, b2_ref, *refs):
    w2_refs = refs[:_S]
    out_ref = refs[_S]
    logits_ref, m_ref, s_ref = refs[_S + 1:]
    t = pl.program_id(0)

    @pl.when(t == 0)
    def _init():
        m_ref[0, 0] = -jnp.inf
        s_ref[0, 0] = 0.0

    @pl.when(t < _P1)
    def _stream():
        h = h_ref[...]
        for k in range(_S):
            cnt = _OFFS[k + 1] - _OFFS[k]

            @pl.when(t < cnt)
            def _do(k=k):
                b = _OFFS[k] + t
                logits = jnp.dot(h, w2_refs[k][...].T,
                                 preferred_element_type=jnp.float32)
                logits = logits + b2_ref[:, pl.ds(b * _R, _R)]
                col = b * _R + jax.lax.broadcasted_iota(jnp.int32, (1, _R), 1)
                logits = jnp.where(col < _V, logits, -jnp.inf)
                logits_ref[:, pl.ds(b * _R, _R)] = logits

                tile_max = jnp.max(logits)
                m_old = m_ref[0, 0]
                m_new = jnp.maximum(m_old, tile_max)
                s_ref[0, 0] = (s_ref[0, 0] * jnp.exp(m_old - m_new)
                               + jnp.sum(jnp.exp(logits - m_new)))
                m_ref[0, 0] = m_new

    @pl.when(t == _P1 - 1)
    def _fin():
        m_ref[0, 0] = m_ref[0, 0] + jnp.log(s_ref[0, 0])

    @pl.when(t == _P1)
    def _emit():
        out_ref[...] = logits_ref[:, :_V] - m_ref[0, 0]


def kernel(inputs, table, W1, b1, W2, b2):
    idx = inputs.astype(jnp.int32)
    b1r = b1.reshape(1, _H)
    b2r = jnp.pad(b2.reshape(1, _V), ((0, 0), (0, _NB * _R - _V)))

    h = pl.pallas_call(
        _l1_kernel,
        in_specs=[
            pl.BlockSpec(memory_space=pltpu.SMEM),
            pl.BlockSpec(memory_space=pl.ANY),
            pl.BlockSpec(memory_space=pltpu.VMEM),
            pl.BlockSpec(memory_space=pltpu.VMEM),
        ],
        out_specs=pl.BlockSpec(memory_space=pltpu.VMEM),
        out_shape=jax.ShapeDtypeStruct((1, _H), jnp.float32),
        scratch_shapes=[
            pltpu.VMEM((1, _CTXW * _D), jnp.float32),
            pltpu.SemaphoreType.DMA((_CTXW,)),
        ],
    )(idx, table, W1, b1r)

    def _w2_map(k):
        lo, hi = _OFFS[k], _OFFS[k + 1] - 1
        return lambda t: (jnp.clip(lo + t, lo, hi), 0)

    out = pl.pallas_call(
        _l2_kernel,
        grid=(_P1 + 1,),
        in_specs=(
            [pl.BlockSpec((1, _H), lambda t: (0, 0)),
             pl.BlockSpec((1, _NB * _R), lambda t: (0, 0))]
            + [pl.BlockSpec((_R, _D), _w2_map(k)) for k in range(_S)]
        ),
        out_specs=pl.BlockSpec((1, _V), lambda t: (0, 0)),
        out_shape=jax.ShapeDtypeStruct((1, _V), jnp.float32),
        scratch_shapes=[
            pltpu.VMEM((1, _NB * _R), jnp.float32),
            pltpu.SMEM((1, 1), jnp.float32),
            pltpu.SMEM((1, 1), jnp.float32),
        ],
    )(h, b2r, *([W2] * _S))

    return out


# speedup vs baseline: 1.7576x; 1.0049x over previous
import jax
import jax.numpy as jnp
from jax.experimental import pallas as pl
from jax.experimental.pallas import tpu as pltpu

_CTXW = 20
_D = 128
_H = 128
_V = 100000
_R = 4096
_NB = (_V + _R - 1) // _R
_S = 5
_P1 = (_NB + _S - 1) // _S
_OFFS = [min(k * _P1, _NB) for k in range(_S + 1)]


def _l1_kernel(idx_ref, tab_ref, w1_ref, b1_ref, h_ref, e_ref, sems):
    for j in range(_CTXW):
        pltpu.make_async_copy(
            tab_ref.at[pl.ds(idx_ref[j], 1), :],
            e_ref.at[:, pl.ds(j * _D, _D)],
            sems.at[j],
        ).start()
    for j in range(_CTXW):
        pltpu.make_async_copy(
            tab_ref.at[pl.ds(idx_ref[j], 1), :],
            e_ref.at[:, pl.ds(j * _D, _D)],
            sems.at[j],
        ).wait()
    h = jnp.dot(e_ref[...], w1_ref[...].T, preferred_element_type=jnp.float32)
    h_ref[...] = jnp.maximum(h + b1_ref[...], 0.0)


def _l2_kernel(h_ref, b2_ref, *refs):
    w2_refs = refs[:_S]
    out_ref = refs[_S]
    logits_ref, m_ref, s_ref = refs[_S + 1:]
    t = pl.program_id(0)

    @pl.when(t == 0)
    def _init():
        m_ref[0, 0] = -jnp.inf
        s_ref[0, 0] = 0.0

    @pl.when(t < _P1)
    def _stream():
        h = h_ref[...]
        for k in range(_S):
            cnt = _OFFS[k + 1] - _OFFS[k]

            @pl.when(t < cnt)
            def _do(k=k):
                b = _OFFS[k] + t
                logits = jnp.dot(h, w2_refs[k][...].T,
                                 preferred_element_type=jnp.float32)
                logits = logits + b2_ref[:, pl.ds(b * _R, _R)]
                col = b * _R + jax.lax.broadcasted_iota(jnp.int32, (1, _R), 1)
                logits = jnp.where(col < _V, logits, -jnp.inf)
                logits_ref[:, pl.ds(b * _R, _R)] = logits

                tile_max = jnp.max(logits)
                m_old = m_ref[0, 0]
                m_new = jnp.maximum(m_old, tile_max)
                s_ref[0, 0] = (s_ref[0, 0] * jnp.exp(m_old - m_new)
                               + jnp.sum(jnp.exp(logits - m_new)))
                m_ref[0, 0] = m_new

    @pl.when(t == _P1 - 1)
    def _fin():
        m_ref[0, 0] = m_ref[0, 0] + jnp.log(s_ref[0, 0])

    @pl.when(t == _P1)
    def _emit():
        out_ref[...] = logits_ref[:, :_V] - m_ref[0, 0]


def kernel(inputs, table, W1, b1, W2, b2):
    idx = inputs.astype(jnp.int32)
    b1r = b1.reshape(1, _H)
    b2r = jnp.pad(b2.reshape(1, _V), ((0, 0), (0, _NB * _R - _V)))

    h = pl.pallas_call(
        _l1_kernel,
        in_specs=[
            pl.BlockSpec(memory_space=pltpu.SMEM),
            pl.BlockSpec(memory_space=pl.ANY),
            pl.BlockSpec(memory_space=pltpu.VMEM),
            pl.BlockSpec(memory_space=pltpu.VMEM),
        ],
        out_specs=pl.BlockSpec(memory_space=pltpu.VMEM),
        out_shape=jax.ShapeDtypeStruct((1, _H), jnp.float32),
        scratch_shapes=[
            pltpu.VMEM((1, _CTXW * _D), jnp.float32),
            pltpu.SemaphoreType.DMA((_CTXW,)),
        ],
    )(idx, table, W1, b1r)

    def _w2_map(k):
        lo, hi = _OFFS[k], _OFFS[k + 1] - 1
        return lambda t: (jnp.clip(lo + t, lo, hi), 0)

    out = pl.pallas_call(
        _l2_kernel,
        grid=(_P1 + 1,),
        in_specs=(
            [pl.BlockSpec((1, _H), lambda t: (0, 0)),
             pl.BlockSpec((1, _NB * _R), lambda t: (0, 0))]
            + [pl.BlockSpec((_R, _D), _w2_map(k)) for k in range(_S)]
        ),
        out_specs=pl.BlockSpec((1, _V), lambda t: (0, 0)),
        out_shape=jax.ShapeDtypeStruct((1, _V), jnp.float32),
        scratch_shapes=[
            pltpu.VMEM((1, _NB * _R), jnp.float32),
            pltpu.SMEM((1, 1), jnp.float32),
            pltpu.SMEM((1, 1), jnp.float32),
        ],
    )(h, b2r, *([W2] * _S))

    return out


# speedup vs baseline: 1.7964x; 1.0221x over previous
import jax
import jax.numpy as jnp
from jax.experimental import pallas as pl
from jax.experimental.pallas import tpu as pltpu

_CTXW = 20
_D = 128
_H = 128
_V = 100000
_R = 6400
_NB = (_V + _R - 1) // _R
_S = 4
_P1 = (_NB + _S - 1) // _S
_OFFS = [min(k * _P1, _NB) for k in range(_S + 1)]


def _l1_kernel(idx_ref, tab_ref, w1_ref, b1_ref, h_ref, e_ref, sems):
    for j in range(_CTXW):
        pltpu.make_async_copy(
            tab_ref.at[pl.ds(idx_ref[j], 1), :],
            e_ref.at[:, pl.ds(j * _D, _D)],
            sems.at[j],
        ).start()
    for j in range(_CTXW):
        pltpu.make_async_copy(
            tab_ref.at[pl.ds(idx_ref[j], 1), :],
            e_ref.at[:, pl.ds(j * _D, _D)],
            sems.at[j],
        ).wait()
    h = jnp.dot(e_ref[...], w1_ref[...].T, preferred_element_type=jnp.float32)
    h_ref[...] = jnp.maximum(h + b1_ref[...], 0.0)


def _l2_kernel(h_ref, b2_ref, *refs):
    w2_refs = refs[:_S]
    out_ref = refs[_S]
    logits_ref, m_ref, s_ref = refs[_S + 1:]
    t = pl.program_id(0)

    @pl.when(t == 0)
    def _init():
        m_ref[0, 0] = -jnp.inf
        s_ref[0, 0] = 0.0

    @pl.when(t < _P1)
    def _stream():
        h = h_ref[...]
        for k in range(_S):
            cnt = _OFFS[k + 1] - _OFFS[k]

            @pl.when(t < cnt)
            def _do(k=k):
                b = _OFFS[k] + t
                logits = jnp.dot(h, w2_refs[k][...].T,
                                 preferred_element_type=jnp.float32)
                logits = logits + b2_ref[:, pl.ds(b * _R, _R)]
                col = b * _R + jax.lax.broadcasted_iota(jnp.int32, (1, _R), 1)
                logits = jnp.where(col < _V, logits, -jnp.inf)
                logits_ref[:, pl.ds(b * _R, _R)] = logits

                tile_max = jnp.max(logits)
                m_old = m_ref[0, 0]
                m_new = jnp.maximum(m_old, tile_max)
                s_ref[0, 0] = (s_ref[0, 0] * jnp.exp(m_old - m_new)
                               + jnp.sum(jnp.exp(logits - m_new)))
                m_ref[0, 0] = m_new

    @pl.when(t == _P1 - 1)
    def _fin():
        m_ref[0, 0] = m_ref[0, 0] + jnp.log(s_ref[0, 0])

    @pl.when(t == _P1)
    def _emit():
        out_ref[...] = logits_ref[:, :_V] - m_ref[0, 0]


def kernel(inputs, table, W1, b1, W2, b2):
    idx = inputs.astype(jnp.int32)
    b1r = b1.reshape(1, _H)
    b2r = jnp.pad(b2.reshape(1, _V), ((0, 0), (0, _NB * _R - _V)))

    h = pl.pallas_call(
        _l1_kernel,
        in_specs=[
            pl.BlockSpec(memory_space=pltpu.SMEM),
            pl.BlockSpec(memory_space=pl.ANY),
            pl.BlockSpec(memory_space=pltpu.VMEM),
            pl.BlockSpec(memory_space=pltpu.VMEM),
        ],
        out_specs=pl.BlockSpec(memory_space=pltpu.VMEM),
        out_shape=jax.ShapeDtypeStruct((1, _H), jnp.float32),
        scratch_shapes=[
            pltpu.VMEM((1, _CTXW * _D), jnp.float32),
            pltpu.SemaphoreType.DMA((_CTXW,)),
        ],
    )(idx, table, W1, b1r)

    def _w2_map(k):
        lo, hi = _OFFS[k], _OFFS[k + 1] - 1
        return lambda t: (jnp.clip(lo + t, lo, hi), 0)

    out = pl.pallas_call(
        _l2_kernel,
        grid=(_P1 + 1,),
        in_specs=(
            [pl.BlockSpec((1, _H), lambda t: (0, 0)),
             pl.BlockSpec((1, _NB * _R), lambda t: (0, 0))]
            + [pl.BlockSpec((_R, _D), _w2_map(k)) for k in range(_S)]
        ),
        out_specs=pl.BlockSpec((1, _V), lambda t: (0, 0)),
        out_shape=jax.ShapeDtypeStruct((1, _V), jnp.float32),
        scratch_shapes=[
            pltpu.VMEM((1, _NB * _R), jnp.float32),
            pltpu.SMEM((1, 1), jnp.float32),
            pltpu.SMEM((1, 1), jnp.float32),
        ],
    )(h, b2r, *([W2] * _S))

    return out
